# baseline (speedup 1.0000x reference)
import functools

import jax
import jax.numpy as jnp
from jax import lax
from jax.experimental import pallas as pl
from jax.experimental.pallas import tpu as pltpu
from jax.experimental.pallas import tpu_sc as plsc

BATCH = 16384
D = 16
NC = 2
NS = 16
NW = NC * NS
BPW = BATCH // NW
GCH = 128
NGC = BPW // GCH
NRC = BPW // 16


def _body(user_hbm, item_i_hbm, item_j_hbm, utab_hbm, itab_hbm, out_hbm,
          uidx, iidx, jidx, urows, irows, jrows, pos_v, neg_v, sem):
    wid = lax.axis_index("s") * NC + lax.axis_index("c")
    base = wid * BPW

    for c in range(NGC):
        off = base + c * GCH
        pltpu.sync_copy(user_hbm.at[pl.ds(off, GCH)], uidx.at[c])
        pltpu.sync_copy(item_i_hbm.at[pl.ds(off, GCH)], iidx.at[c])
        pltpu.sync_copy(item_j_hbm.at[pl.ds(off, GCH)], jidx.at[c])

    copies = []
    for c in range(NGC):
        dst = pl.ds(c * GCH, GCH)
        copies.append(pltpu.async_copy(utab_hbm.at[uidx.at[c]], urows.at[dst], sem))
        copies.append(pltpu.async_copy(itab_hbm.at[iidx.at[c]], irows.at[dst], sem))
        copies.append(pltpu.async_copy(itab_hbm.at[jidx.at[c]], jrows.at[dst], sem))
    for cp in copies:
        cp.wait()

    lane = lax.iota(jnp.int32, 16)

    def chunk(t, carry):
        rows = t * 16 + lane
        accp = jnp.zeros((16,), jnp.float32)
        accn = jnp.zeros((16,), jnp.float32)
        for d in range(D):
            dv = jnp.full((16,), d, jnp.int32)
            u = plsc.load_gather(urows, [rows, dv])
            vi = plsc.load_gather(irows, [rows, dv])
            vj = plsc.load_gather(jrows, [rows, dv])
            accp = accp + u * vi
            accn = accn + u * vj
        pos_v[pl.ds(t * 16, 16)] = accp
        neg_v[pl.ds(t * 16, 16)] = accn
        return carry

    lax.fori_loop(0, NRC, chunk, 0)

    pltpu.sync_copy(pos_v, out_hbm.at[pl.ds(base, BPW)])
    pltpu.sync_copy(neg_v, out_hbm.at[pl.ds(BATCH + base, BPW)])


def kernel(user, item_i, item_j, user_table, item_table):
    user = user.astype(jnp.int32)
    item_i = item_i.astype(jnp.int32)
    item_j = item_j.astype(jnp.int32)
    mesh = plsc.VectorSubcoreMesh(core_axis_name="c", subcore_axis_name="s")
    run = functools.partial(
        pl.kernel,
        mesh=mesh,
        compiler_params=pltpu.CompilerParams(
            needs_layout_passes=False, use_tc_tiling_on_sc=False),
        out_type=jax.ShapeDtypeStruct((2 * BATCH,), jnp.float32),
        scratch_types=[
            pltpu.VMEM((NGC, GCH), jnp.int32),
            pltpu.VMEM((NGC, GCH), jnp.int32),
            pltpu.VMEM((NGC, GCH), jnp.int32),
            pltpu.VMEM((BPW, D), jnp.float32),
            pltpu.VMEM((BPW, D), jnp.float32),
            pltpu.VMEM((BPW, D), jnp.float32),
            pltpu.VMEM((BPW,), jnp.float32),
            pltpu.VMEM((BPW,), jnp.float32),
            pltpu.SemaphoreType.DMA,
        ],
    )(_body)
    return run(user, item_i, item_j, user_table, item_table)


# speedup vs baseline: 1.4860x; 1.4860x over previous
import functools

import jax
import jax.numpy as jnp
from jax import lax
from jax.experimental import pallas as pl
from jax.experimental.pallas import tpu as pltpu
from jax.experimental.pallas import tpu_sc as plsc

BATCH = 16384
D = 16
NC = 2
NS = 16
NW = NC * NS
BPW = BATCH // NW
NRC = BPW // 16
NBUF = 2


def _body(user_hbm, item_i_hbm, item_j_hbm, utab_hbm, itab_hbm, out_hbm,
          uidx, iidx, jidx, urows, irows, jrows, pos_v, neg_v, sems):
    wid = lax.axis_index("s") * NC + lax.axis_index("c")
    base = wid * BPW

    pltpu.sync_copy(user_hbm.at[pl.ds(base, BPW)], uidx)
    pltpu.sync_copy(item_i_hbm.at[pl.ds(base, BPW)], iidx)
    pltpu.sync_copy(item_j_hbm.at[pl.ds(base, BPW)], jidx)

    tabs = ((utab_hbm, uidx, urows), (itab_hbm, iidx, irows),
            (itab_hbm, jidx, jrows))

    def fire(t, b):
        for tab, ibuf, rbuf in tabs:
            v = ibuf[pl.ds(t * 16, 16)]
            for i in range(16):
                pltpu.async_copy(
                    tab.at[pl.ds(v[i], 1), :],
                    rbuf.at[pl.ds(b * 16 + i, 1), :],
                    sems.at[b],
                )

    def drain(b):
        for tab, _, rbuf in tabs:
            pltpu.make_async_copy(
                tab.at[pl.ds(0, 16), :], rbuf.at[pl.ds(b * 16, 16), :],
                sems.at[b],
            ).wait()

    lane = lax.iota(jnp.int32, 16)

    def compute(t, b):
        r0 = b * 16
        accp = jnp.zeros((16,), jnp.float32)
        accn = jnp.zeros((16,), jnp.float32)
        for i in range(16):
            u = urows[r0 + i]
            vi = irows[r0 + i]
            vj = jrows[r0 + i]
            sp = jnp.sum(u * vi)
            sn = jnp.sum(u * vj)
            m = lane == i
            accp = jnp.where(m, sp, accp)
            accn = jnp.where(m, sn, accn)
        pos_v[pl.ds(t * 16, 16)] = accp
        neg_v[pl.ds(t * 16, 16)] = accn

    fire(0, 0)

    def body(t, carry):
        b = lax.rem(t, NBUF)

        @pl.when(t + 1 < NRC)
        def _():
            fire(t + 1, lax.rem(t + 1, NBUF))

        drain(b)
        compute(t, b)
        return carry

    lax.fori_loop(0, NRC, body, 0)

    pltpu.sync_copy(pos_v, out_hbm.at[pl.ds(base, BPW)])
    pltpu.sync_copy(neg_v, out_hbm.at[pl.ds(BATCH + base, BPW)])


def kernel(user, item_i, item_j, user_table, item_table):
    user = user.astype(jnp.int32)
    item_i = item_i.astype(jnp.int32)
    item_j = item_j.astype(jnp.int32)
    mesh = plsc.VectorSubcoreMesh(core_axis_name="c", subcore_axis_name="s")
    run = functools.partial(
        pl.kernel,
        mesh=mesh,
        compiler_params=pltpu.CompilerParams(needs_layout_passes=False),
        out_type=jax.ShapeDtypeStruct((2 * BATCH,), jnp.float32),
        scratch_types=[
            pltpu.VMEM((BPW,), jnp.int32),
            pltpu.VMEM((BPW,), jnp.int32),
            pltpu.VMEM((BPW,), jnp.int32),
            pltpu.VMEM((NBUF * 16, D), jnp.float32),
            pltpu.VMEM((NBUF * 16, D), jnp.float32),
            pltpu.VMEM((NBUF * 16, D), jnp.float32),
            pltpu.VMEM((BPW,), jnp.float32),
            pltpu.VMEM((BPW,), jnp.float32),
            pltpu.SemaphoreType.DMA((NBUF,)),
        ],
    )(_body)
    return run(user, item_i, item_j, user_table, item_table)
